# Initial kernel scaffold; baseline (speedup 1.0000x reference)
import jax
import jax.numpy as jnp
from jax.experimental import pallas as pl


def kernel(i_indices, j_indices, pe):
    raise NotImplementedError("write your pallas kernel here")


# speedup vs baseline: 17.3831x; 17.3831x over previous
import functools

import jax
import jax.numpy as jnp
from jax import lax
from jax.experimental import pallas as pl
from jax.experimental.pallas import tpu as pltpu
from jax.experimental.pallas import tpu_sc as plsc

D_HALF = 64
_NC = 2
_NS = 16
_NW = _NC * _NS
_CHUNK = 512
_LANES = 16


@functools.partial(jax.jit, static_argnames=("n",))
def _sc_gather_pairs(i_flat, j_flat, ptab, n):
    per_w = n // _NW
    steps = per_w // _CHUNK
    mesh = plsc.VectorSubcoreMesh(core_axis_name="c", subcore_axis_name="s")

    @functools.partial(
        pl.kernel,
        mesh=mesh,
        out_type=jax.ShapeDtypeStruct((n, 2 * D_HALF), jnp.float32),
        scratch_types=[
            pltpu.VMEM((_CHUNK,), jnp.int32),
            pltpu.VMEM((_CHUNK,), jnp.int32),
            pltpu.VMEM((_CHUNK,), jnp.int32),
            pltpu.VMEM((_CHUNK, 2 * D_HALF), jnp.float32),
            pltpu.SemaphoreType.DMA,
        ],
    )
    def k(i_hbm, j_hbm, ptab_hbm, out_hbm, iv, jv, kv, rows, sem):
        wid = lax.axis_index("s") * _NC + lax.axis_index("c")
        wbase = wid * per_w

        def body(t, carry):
            base = wbase + t * _CHUNK
            pltpu.sync_copy(i_hbm.at[pl.ds(base, _CHUNK)], iv)
            pltpu.sync_copy(j_hbm.at[pl.ds(base, _CHUNK)], jv)

            def combine(v, c):
                sl = pl.ds(v * _LANES, _LANES)
                kv[sl] = iv[sl] * 100 + jv[sl]
                return c

            lax.fori_loop(0, _CHUNK // _LANES, combine, 0)
            pltpu.async_copy(ptab_hbm.at[kv], rows, sem).wait()
            pltpu.sync_copy(rows, out_hbm.at[pl.ds(base, _CHUNK)])
            return carry

        lax.fori_loop(0, steps, body, 0)

    return k(i_flat, j_flat, ptab)


def kernel(i_indices, j_indices, pe):
    b, p = i_indices.shape
    n = b * p
    v = pe.shape[0]
    ptab = jnp.concatenate(
        [jnp.repeat(pe, v, axis=0), jnp.tile(pe, (v, 1))], axis=1
    )
    out = _sc_gather_pairs(i_indices.reshape(-1), j_indices.reshape(-1), ptab, n)
    return out.reshape(b, p, 2 * D_HALF)


# speedup vs baseline: 21.6799x; 1.2472x over previous
import functools

import jax
import jax.numpy as jnp
from jax import lax
from jax.experimental import pallas as pl
from jax.experimental.pallas import tpu as pltpu
from jax.experimental.pallas import tpu_sc as plsc

D_HALF = 64
D = 2 * D_HALF
_NC = 2
_NS = 16
_NW = _NC * _NS
_CHUNK = 400
_LANES = 16


@functools.partial(jax.jit, static_argnames=("n",))
def _sc_gather_pairs(i_flat, j_flat, ptab, n):
    per_w = n // _NW
    steps = per_w // _CHUNK
    assert steps % 2 == 0
    mesh = plsc.VectorSubcoreMesh(core_axis_name="c", subcore_axis_name="s")

    @functools.partial(
        pl.kernel,
        mesh=mesh,
        out_type=jax.ShapeDtypeStruct((n, D), jnp.float32),
        scratch_types=[
            pltpu.VMEM((_CHUNK,), jnp.int32),
            pltpu.VMEM((_CHUNK,), jnp.int32),
            pltpu.VMEM((_CHUNK,), jnp.int32),
            pltpu.VMEM((_CHUNK,), jnp.int32),
            pltpu.VMEM((_CHUNK,), jnp.int32),
            pltpu.VMEM((_CHUNK,), jnp.int32),
            pltpu.VMEM((2, _CHUNK, D), jnp.float32),
            pltpu.SemaphoreType.DMA,
            pltpu.SemaphoreType.DMA,
            pltpu.SemaphoreType.DMA,
            pltpu.SemaphoreType.DMA,
            pltpu.SemaphoreType.DMA,
            pltpu.SemaphoreType.DMA,
        ],
    )
    def k(i_hbm, j_hbm, ptab_hbm, out_hbm, iv0, iv1, jv0, jv1, kv0, kv1,
          rows, sg0, sg1, sw0, sw1, si0, si1):
        iv = (iv0, iv1)
        jv = (jv0, jv1)
        kv = (kv0, kv1)
        sg = (sg0, sg1)
        sw = (sw0, sw1)
        si = (si0, si1)
        wid = lax.axis_index("s") * _NC + lax.axis_index("c")
        wbase = wid * per_w

        def idx_start(t, s):
            base = wbase + t * _CHUNK
            pltpu.async_copy(i_hbm.at[pl.ds(base, _CHUNK)], iv[s], si[s])
            pltpu.async_copy(j_hbm.at[pl.ds(base, _CHUNK)], jv[s], si[s])

        def idx_wait(t, s):
            base = wbase + t * _CHUNK
            pltpu.make_async_copy(
                i_hbm.at[pl.ds(base, _CHUNK)], iv[s], si[s]).wait()
            pltpu.make_async_copy(
                j_hbm.at[pl.ds(base, _CHUNK)], jv[s], si[s]).wait()

        def combine(s):
            def body(v, c):
                sl = pl.ds(v * _LANES, _LANES)
                kv[s][sl] = iv[s][sl] * 100 + jv[s][sl]
                return c
            lax.fori_loop(0, _CHUNK // _LANES, body, 0)

        def gather_start(s):
            pltpu.async_copy(ptab_hbm.at[kv[s]], rows.at[s], sg[s])

        def gather_wait(s):
            pltpu.make_async_copy(
                ptab_hbm.at[kv[s]], rows.at[s], sg[s]).wait()

        def write_start(t, s):
            base = wbase + t * _CHUNK
            pltpu.async_copy(rows.at[s], out_hbm.at[pl.ds(base, _CHUNK)], sw[s])

        def write_wait(t, s):
            base = wbase + t * _CHUNK
            pltpu.make_async_copy(
                rows.at[s], out_hbm.at[pl.ds(base, _CHUNK)], sw[s]).wait()

        idx_start(0, 0)
        idx_wait(0, 0)
        combine(0)
        gather_start(0)
        idx_start(1, 1)

        def outer(g, carry):
            for s in (0, 1):
                t = 2 * g + s
                gather_wait(s)
                write_start(t, s)
                ns = 1 - s

                @pl.when(t + 1 < steps)
                def _():
                    idx_wait(t + 1, ns)
                    combine(ns)

                    @pl.when(t >= 1)
                    def _():
                        write_wait(t - 1, ns)

                    gather_start(ns)

                    @pl.when(t + 2 < steps)
                    def _():
                        idx_start(t + 2, s)
            return carry

        lax.fori_loop(0, steps // 2, outer, 0)
        write_wait(steps - 2, 0)
        write_wait(steps - 1, 1)

    return k(i_flat, j_flat, ptab)


def kernel(i_indices, j_indices, pe):
    b, p = i_indices.shape
    n = b * p
    v = pe.shape[0]
    ptab = jnp.concatenate(
        [jnp.repeat(pe, v, axis=0), jnp.tile(pe, (v, 1))], axis=1
    )
    out = _sc_gather_pairs(i_indices.reshape(-1), j_indices.reshape(-1), ptab, n)
    return out.reshape(b, p, D)


# speedup vs baseline: 36.3180x; 1.6752x over previous
import functools

import jax
import jax.numpy as jnp
from jax import lax
from jax.experimental import pallas as pl
from jax.experimental.pallas import tpu as pltpu
from jax.experimental.pallas import tpu_sc as plsc

D_HALF = 64
D = 2 * D_HALF
_NC = 2
_NS = 16
_NW = _NC * _NS
_CHUNK = 160
_LANES = 16


@functools.partial(jax.jit, static_argnames=("n",))
def _sc_gather_pairs(i_flat, j_flat, ptab, n):
    per_w = n // _NW
    steps = per_w // _CHUNK
    assert steps % 2 == 0
    mesh = plsc.VectorSubcoreMesh(core_axis_name="c", subcore_axis_name="s")

    @functools.partial(
        pl.kernel,
        mesh=mesh,
        out_type=jax.ShapeDtypeStruct((n, D), jnp.float32),
        scratch_types=[
            pltpu.VMEM((_CHUNK,), jnp.int32),
            pltpu.VMEM((_CHUNK,), jnp.int32),
            pltpu.VMEM((_CHUNK,), jnp.int32),
            pltpu.VMEM((_CHUNK,), jnp.int32),
            pltpu.VMEM((_CHUNK,), jnp.int32),
            pltpu.VMEM((_CHUNK,), jnp.int32),
            pltpu.VMEM((2, _CHUNK, D), jnp.float32),
            pltpu.VMEM_SHARED((10000, D), jnp.float32),
            pltpu.SemaphoreType.DMA,
            pltpu.SemaphoreType.DMA,
            pltpu.SemaphoreType.DMA,
            pltpu.SemaphoreType.DMA,
            pltpu.SemaphoreType.DMA,
            pltpu.SemaphoreType.DMA,
        ],
    )
    def k(i_hbm, j_hbm, ptab_hbm, out_hbm, iv0, iv1, jv0, jv1, kv0, kv1,
          rows, ptab_sp, sg0, sg1, sw0, sw1, si0, si1):
        iv = (iv0, iv1)
        jv = (jv0, jv1)
        kv = (kv0, kv1)
        sg = (sg0, sg1)
        sw = (sw0, sw1)
        si = (si0, si1)
        wid = lax.axis_index("s") * _NC + lax.axis_index("c")
        wbase = wid * per_w

        def idx_start(t, s):
            base = wbase + t * _CHUNK
            pltpu.async_copy(i_hbm.at[pl.ds(base, _CHUNK)], iv[s], si[s])
            pltpu.async_copy(j_hbm.at[pl.ds(base, _CHUNK)], jv[s], si[s])

        def idx_wait(t, s):
            base = wbase + t * _CHUNK
            pltpu.make_async_copy(
                i_hbm.at[pl.ds(base, _CHUNK)], iv[s], si[s]).wait()
            pltpu.make_async_copy(
                j_hbm.at[pl.ds(base, _CHUNK)], jv[s], si[s]).wait()

        def combine(s):
            def body(v, c):
                sl = pl.ds(v * _LANES, _LANES)
                kv[s][sl] = iv[s][sl] * 100 + jv[s][sl]
                return c
            lax.fori_loop(0, _CHUNK // _LANES, body, 0)

        def gather_start(s):
            pltpu.async_copy(ptab_sp.at[kv[s]], rows.at[s], sg[s])

        def gather_wait(s):
            pltpu.make_async_copy(
                ptab_sp.at[kv[s]], rows.at[s], sg[s]).wait()

        def write_start(t, s):
            base = wbase + t * _CHUNK
            pltpu.async_copy(rows.at[s], out_hbm.at[pl.ds(base, _CHUNK)], sw[s])

        def write_wait(t, s):
            base = wbase + t * _CHUNK
            pltpu.make_async_copy(
                rows.at[s], out_hbm.at[pl.ds(base, _CHUNK)], sw[s]).wait()

        @pl.when(lax.axis_index("s") == 0)
        def _():
            pltpu.sync_copy(ptab_hbm, ptab_sp)

        plsc.subcore_barrier()

        idx_start(0, 0)
        idx_wait(0, 0)
        combine(0)
        gather_start(0)
        idx_start(1, 1)

        def outer(g, carry):
            for s in (0, 1):
                t = 2 * g + s
                gather_wait(s)
                write_start(t, s)
                ns = 1 - s

                @pl.when(t + 1 < steps)
                def _():
                    idx_wait(t + 1, ns)
                    combine(ns)

                    @pl.when(t >= 1)
                    def _():
                        write_wait(t - 1, ns)

                    gather_start(ns)

                    @pl.when(t + 2 < steps)
                    def _():
                        idx_start(t + 2, s)
            return carry

        lax.fori_loop(0, steps // 2, outer, 0)
        write_wait(steps - 2, 0)
        write_wait(steps - 1, 1)

    return k(i_flat, j_flat, ptab)


def kernel(i_indices, j_indices, pe):
    b, p = i_indices.shape
    n = b * p
    v = pe.shape[0]
    ptab = jnp.concatenate(
        [jnp.repeat(pe, v, axis=0), jnp.tile(pe, (v, 1))], axis=1
    )
    out = _sc_gather_pairs(i_indices.reshape(-1), j_indices.reshape(-1), ptab, n)
    return out.reshape(b, p, D)
